# baseline (speedup 1.0000x reference)
import functools

import jax
import jax.numpy as jnp
from jax import lax
from jax.experimental import pallas as pl
from jax.experimental.pallas import tpu as pltpu
from jax.experimental.pallas import tpu_sc as plsc

_DIM = 64
_LANES = 16
_CHUNK = 128


@functools.lru_cache(maxsize=None)
def _build(batch, vocab, dim):
    info = plsc.get_sparse_core_info()
    nc, ns = info.num_cores, info.num_subcores
    nw = nc * ns
    b_per_w = batch // nw
    nch = b_per_w // _CHUNK
    nslice = dim // _LANES

    mesh = plsc.VectorSubcoreMesh(core_axis_name="c", subcore_axis_name="s")

    @functools.partial(
        pl.kernel,
        mesh=mesh,
        out_type=jax.ShapeDtypeStruct((batch,), jnp.float32),
        compiler_params=pltpu.CompilerParams(
            needs_layout_passes=False, use_tc_tiling_on_sc=False),
        scratch_types=[
            pltpu.VMEM((nch, _CHUNK), jnp.int32),
            pltpu.VMEM((nch, _CHUNK), jnp.int32),
            pltpu.VMEM((2, _CHUNK, dim), jnp.float32),
            pltpu.VMEM((2, _CHUNK, dim), jnp.float32),
            pltpu.VMEM((b_per_w,), jnp.float32),
            pltpu.SemaphoreType.DMA,
            pltpu.SemaphoreType.DMA,
            pltpu.SemaphoreType.DMA,
            pltpu.SemaphoreType.DMA,
        ],
    )
    def word2vec_sc(emb_hbm, oemb_hbm, ci_hbm, xi_hbm, out_hbm,
                    ci_v, xi_v, a_v, b_v, o_v, sa0, sa1, sb0, sb1):
        wid = lax.axis_index("s") * nc + lax.axis_index("c")
        base = wid * b_per_w
        pltpu.sync_copy(ci_hbm.at[wid], ci_v)
        pltpu.sync_copy(xi_hbm.at[wid], xi_v)
        sas = (sa0, sa1)
        sbs = (sb0, sb1)

        def start(j):
            buf = j % 2
            ha = pltpu.async_copy(emb_hbm.at[ci_v.at[j]], a_v.at[buf], sas[buf])
            hb = pltpu.async_copy(oemb_hbm.at[xi_v.at[j]], b_v.at[buf], sbs[buf])
            return ha, hb

        handles = [None] * nch
        handles[0] = start(0)
        for j in range(nch):
            if j + 1 < nch:
                handles[j + 1] = start(j + 1)
            ha, hb = handles[j]
            ha.wait()
            hb.wait()
            buf = j % 2

            lane_iota = jnp.arange(_LANES, dtype=jnp.int32)

            def grp(g, carry, _buf=buf, _j=j):
                out = jnp.zeros((_LANES,), dtype=jnp.float32)
                for rr in range(_LANES):
                    r = g * _LANES + rr
                    acc = a_v[_buf, r, 0:_LANES] * b_v[_buf, r, 0:_LANES]
                    for c in range(1, nslice):
                        lo = c * _LANES
                        acc = acc + (a_v[_buf, r, lo:lo + _LANES]
                                     * b_v[_buf, r, lo:lo + _LANES])
                    s = jnp.sum(acc)
                    out = jnp.where(lane_iota == rr, s, out)
                o_v[pl.ds(_j * _CHUNK + g * _LANES, _LANES)] = out
                return carry

            lax.fori_loop(0, _CHUNK // _LANES, grp, 0)

        pltpu.sync_copy(o_v, out_hbm.at[pl.ds(base, b_per_w)])

    return word2vec_sc


def kernel(center, context, embeddings, output_embeddings):
    batch = center.shape[0]
    vocab, dim = embeddings.shape
    info = plsc.get_sparse_core_info()
    nw = info.num_cores * info.num_subcores
    b_per_w = batch // nw
    nch = b_per_w // _CHUNK
    ci = center.astype(jnp.int32).reshape(nw, nch, _CHUNK)
    xi = context.astype(jnp.int32).reshape(nw, nch, _CHUNK)
    fn = _build(batch, vocab, dim)
    return fn(embeddings, output_embeddings, ci, xi)
